# Initial kernel scaffold; baseline (speedup 1.0000x reference)
import jax
import jax.numpy as jnp
from jax.experimental import pallas as pl


def kernel(x, edge_attr, edge_index, batch, params):
    raise NotImplementedError("write your pallas kernel here")


# speedup vs baseline: 2.2063x; 2.2063x over previous
import functools

import jax
import jax.numpy as jnp
from jax import lax
from jax.experimental import pallas as pl
from jax.experimental.pallas import tpu as pltpu
from jax.experimental.pallas import tpu_sc as plsc

N = 10000
E = 160000
D = 128
OUT = 3

NW = 32
CHUNK = 128
EP = 163840
CPW = EP // (NW * CHUNK)
NP2 = 10112
PAD_SINK = N

BR = 2000


def _ln(v, g, b):
    mu = jnp.mean(v, axis=-1, keepdims=True)
    vc = v - mu
    var = jnp.mean(vc * vc, axis=-1, keepdims=True)
    return vc * lax.rsqrt(var + 1e-5) * g + b


def _row_spec(i_map=None):
    return pl.BlockSpec((BR, D), i_map or (lambda i: (i, 0)))


def _full_spec(shape):
    return pl.BlockSpec(shape, lambda i: tuple(0 for _ in shape))


def _enc_body(x_ref, w0, b0, w1, b1, g, b, o_ref):
    x = x_ref[...]
    z = jnp.maximum(jnp.dot(x, w0[...], preferred_element_type=jnp.float32) + b0[...], 0.0)
    v = jnp.dot(z, w1[...], preferred_element_type=jnp.float32) + b1[...]
    o_ref[...] = _ln(v, g[...], b[...])


def _tc_encoder(x, w0, b0, w1, b1, g, b):
    rows = x.shape[0]
    grid = (rows // BR,)
    return pl.pallas_call(
        _enc_body,
        grid=grid,
        in_specs=[_row_spec(), _full_spec((D, D)), _full_spec((1, D)),
                  _full_spec((D, D)), _full_spec((1, D)),
                  _full_spec((1, D)), _full_spec((1, D))],
        out_specs=_row_spec(),
        out_shape=jax.ShapeDtypeStruct((rows, D), jnp.float32),
    )(x, w0, b0, w1, b1, g, b)


def _proj_body(h_ref, wd, bd, ws, wh, bh, pd_ref, ps_ref, ph_ref):
    h = h_ref[...]
    pd_ref[...] = jnp.dot(h, wd[...], preferred_element_type=jnp.float32) + bd[...]
    ps_ref[...] = jnp.dot(h, ws[...], preferred_element_type=jnp.float32)
    ph_ref[...] = jnp.dot(h, wh[...], preferred_element_type=jnp.float32) + bh[...]


def _tc_nodeproj(h, wd, bd, ws, wh, bh):
    grid = (N // BR,)
    return pl.pallas_call(
        _proj_body,
        grid=grid,
        in_specs=[_row_spec(), _full_spec((D, D)), _full_spec((1, D)),
                  _full_spec((D, D)), _full_spec((D, D)), _full_spec((1, D))],
        out_specs=[_row_spec(), _row_spec(), _row_spec()],
        out_shape=[jax.ShapeDtypeStruct((N, D), jnp.float32)] * 3,
    )(h, wd, bd, ws, wh, bh)


def _edge_body(gd_ref, gs_ref, e_ref, we, w2, b2, g, b, ne_ref, eo_ref):
    e = e_ref[...]
    z = gd_ref[...] + gs_ref[...] + jnp.dot(e, we[...], preferred_element_type=jnp.float32)
    u = jnp.maximum(z, 0.0)
    v = jnp.dot(u, w2[...], preferred_element_type=jnp.float32) + b2[...]
    ne = _ln(v, g[...], b[...])
    ne_ref[...] = ne
    eo_ref[...] = e + ne


def _tc_edge(gd, gs, e, we, w2, b2, g, b):
    grid = (E // BR,)
    return pl.pallas_call(
        _edge_body,
        grid=grid,
        in_specs=[_row_spec(), _row_spec(), _row_spec(),
                  _full_spec((D, D)), _full_spec((D, D)), _full_spec((1, D)),
                  _full_spec((1, D)), _full_spec((1, D))],
        out_specs=[_row_spec(), _row_spec()],
        out_shape=[jax.ShapeDtypeStruct((EP, D), jnp.float32),
                   jax.ShapeDtypeStruct((E, D), jnp.float32)],
    )(gd, gs, e, we, w2, b2, g, b)


def _upd_body(h_ref, ph_ref, agg_ref, wa, w2, b2, g, b, ho_ref):
    agg = agg_ref[0] + agg_ref[1]
    z = ph_ref[...] + jnp.dot(agg, wa[...], preferred_element_type=jnp.float32)
    u = jnp.maximum(z, 0.0)
    v = jnp.dot(u, w2[...], preferred_element_type=jnp.float32) + b2[...]
    ho_ref[...] = h_ref[...] + _ln(v, g[...], b[...])


def _tc_nodeupd(h, ph, aggp, wa, w2, b2, g, b):
    grid = (N // BR,)
    return pl.pallas_call(
        _upd_body,
        grid=grid,
        in_specs=[_row_spec(), _row_spec(),
                  pl.BlockSpec((2, BR, D), lambda i: (0, i, 0)),
                  _full_spec((D, D)), _full_spec((D, D)), _full_spec((1, D)),
                  _full_spec((1, D)), _full_spec((1, D))],
        out_specs=_row_spec(),
        out_shape=jax.ShapeDtypeStruct((N, D), jnp.float32),
    )(h, ph, aggp, wa, w2, b2, g, b)


def _dec_body(h_ref, w0, b0, w1, b1, o_ref):
    h = h_ref[...]
    z = jnp.maximum(jnp.dot(h, w0[...], preferred_element_type=jnp.float32) + b0[...], 0.0)
    o_ref[...] = jnp.dot(z, w1[...], preferred_element_type=jnp.float32) + b1[...]


def _tc_decoder(h, w0, b0, w1, b1):
    grid = (N // BR,)
    return pl.pallas_call(
        _dec_body,
        grid=grid,
        in_specs=[_row_spec(), _full_spec((D, D)), _full_spec((1, D)),
                  _full_spec((D, D)), _full_spec((1, D))],
        out_specs=_row_spec(),
        out_shape=jax.ShapeDtypeStruct((N, D), jnp.float32),
    )(h, w0, b0, w1, b1)


def _gather_body(pd, ps, dsti, srci, gd, gs, idxd, idxs, bufd, bufs, semd, sems):
    c = lax.axis_index("c")
    s = lax.axis_index("s")
    w = c * 16 + s
    row0 = w * CPW
    pltpu.sync_copy(dsti.at[pl.ds(row0, CPW)], idxd)
    pltpu.sync_copy(srci.at[pl.ds(row0, CPW)], idxs)

    def body(j, carry):
        cpd = pltpu.async_copy(pd.at[idxd.at[j]], bufd, semd)
        cps = pltpu.async_copy(ps.at[idxs.at[j]], bufs, sems)
        cpd.wait()
        cps.wait()
        erow = pl.multiple_of((row0 + j) * CHUNK, CHUNK)
        pltpu.sync_copy(bufd, gd.at[pl.ds(erow, CHUNK)])
        pltpu.sync_copy(bufs, gs.at[pl.ds(erow, CHUNK)])
        return carry

    lax.fori_loop(0, CPW, body, 0)


@functools.cache
def _sc_gather_kernel():
    return pl.kernel(
        _gather_body,
        out_type=[jax.ShapeDtypeStruct((EP, D), jnp.float32),
                  jax.ShapeDtypeStruct((EP, D), jnp.float32)],
        mesh=plsc.VectorSubcoreMesh(core_axis_name="c", subcore_axis_name="s"),
        scratch_types=[pltpu.VMEM((CPW, CHUNK), jnp.int32),
                       pltpu.VMEM((CPW, CHUNK), jnp.int32),
                       pltpu.VMEM((CHUNK, D), jnp.float32),
                       pltpu.VMEM((CHUNK, D), jnp.float32),
                       pltpu.SemaphoreType.DMA,
                       pltpu.SemaphoreType.DMA],
    )


def _sc_gather(pd, ps, dsti, srci):
    return _sc_gather_kernel()(pd, ps, dsti, srci)


def _scatter_body(ne, dsti, zz, out, idx, ebuf, acc):
    c = lax.axis_index("c")
    s = lax.axis_index("s")
    w = c * 16 + s
    row0 = w * CPW
    pltpu.sync_copy(zz.at[pl.ds(s * 632, 632)], acc.at[pl.ds(s * 632, 632)])
    pltpu.sync_copy(dsti.at[pl.ds(row0, CPW)], idx)
    plsc.subcore_barrier()

    def body(j, carry):
        erow = pl.multiple_of((row0 + j) * CHUNK, CHUNK)
        pltpu.sync_copy(ne.at[pl.ds(erow, CHUNK)], ebuf)
        pltpu.sync_copy(ebuf, acc.at[idx.at[j]], add=True)
        return carry

    lax.fori_loop(0, CPW, body, 0)
    plsc.subcore_barrier()

    @pl.when(s < 15)
    def _():
        pltpu.sync_copy(acc.at[pl.ds(s * 632, 632)], out.at[c, pl.ds(s * 632, 632)])

    @pl.when(s == 15)
    def _():
        pltpu.sync_copy(acc.at[pl.ds(15 * 632, N - 15 * 632)],
                        out.at[c, pl.ds(15 * 632, N - 15 * 632)])


@functools.cache
def _sc_scatter_kernel():
    return pl.kernel(
        _scatter_body,
        out_type=jax.ShapeDtypeStruct((2, N, D), jnp.float32),
        mesh=plsc.VectorSubcoreMesh(core_axis_name="c", subcore_axis_name="s"),
        scratch_types=[pltpu.VMEM((CPW, CHUNK), jnp.int32),
                       pltpu.VMEM((CHUNK, D), jnp.float32),
                       pltpu.VMEM_SHARED((NP2, D), jnp.float32)],
    )


def _sc_scatter(ne, dsti, zz):
    return _sc_scatter_kernel()(ne, dsti, zz)


def _r1(v):
    return v.reshape(1, D)


def kernel(x, edge_attr, edge_index, batch, params):
    src = edge_index[0]
    dst = edge_index[1]

    padz = jnp.zeros((EP - E,), jnp.int32)
    dst_g = jnp.concatenate([dst, padz]).reshape(EP // CHUNK, CHUNK)
    src_g = jnp.concatenate([src, padz]).reshape(EP // CHUNK, CHUNK)
    dst_s = jnp.concatenate([dst, jnp.full((EP - E,), PAD_SINK, jnp.int32)]
                            ).reshape(EP // CHUNK, CHUNK)
    zeros_acc = jnp.zeros((NP2, D), jnp.float32)

    en = params["enc_node"]
    ee = params["enc_edge"]
    h = _tc_encoder(x, en["W"][0], _r1(en["b"][0]), en["W"][1], _r1(en["b"][1]),
                    _r1(en["ln_g"]), _r1(en["ln_b"]))
    e = _tc_encoder(edge_attr, ee["W"][0], _r1(ee["b"][0]), ee["W"][1], _r1(ee["b"][1]),
                    _r1(ee["ln_g"]), _r1(ee["ln_b"]))

    for blk in params["blocks"]:
        m, u = blk["msg"], blk["upd"]
        w1m = m["W"][0]
        wd, ws, we = w1m[:D], w1m[D:2 * D], w1m[2 * D:]
        w1u = u["W"][0]
        wh, wa = w1u[:D], w1u[D:]

        pd, ps, ph = _tc_nodeproj(h, wd, _r1(m["b"][0]), ws, wh, _r1(u["b"][0]))
        gd, gs = _sc_gather(pd, ps, dst_g, src_g)
        ne, e = _tc_edge(gd, gs, e, we, m["W"][1], _r1(m["b"][1]),
                         _r1(m["ln_g"]), _r1(m["ln_b"]))
        aggp = _sc_scatter(ne, dst_s, zeros_acc)
        h = _tc_nodeupd(h, ph, aggp, wa, u["W"][1], _r1(u["b"][1]),
                        _r1(u["ln_g"]), _r1(u["ln_b"]))

    dec = params["dec"]
    w0 = jnp.pad(dec["W"][0], ((0, 0), (0, D - OUT)))
    b0 = _r1(jnp.pad(dec["b"][0], (0, D - OUT)))
    w1 = jnp.pad(dec["W"][1], ((0, D - OUT), (0, D - OUT)))
    b1 = _r1(jnp.pad(dec["b"][1], (0, D - OUT)))
    out_full = _tc_decoder(h, w0, b0, w1, b1)
    return out_full[:, :OUT]


# speedup vs baseline: 2.4646x; 1.1171x over previous
import functools

import jax
import jax.numpy as jnp
from jax import lax
from jax.experimental import pallas as pl
from jax.experimental.pallas import tpu as pltpu
from jax.experimental.pallas import tpu_sc as plsc

N = 10000
E = 160000
D = 128
OUT = 3

NW = 32
CHUNK = 128
EP = 163840
CPW = EP // (NW * CHUNK)
NP2 = 10112
PAD_SINK = N

BR = 2000


def _ln(v, g, b):
    mu = jnp.mean(v, axis=-1, keepdims=True)
    vc = v - mu
    var = jnp.mean(vc * vc, axis=-1, keepdims=True)
    return vc * lax.rsqrt(var + 1e-5) * g + b


def _row_spec(i_map=None):
    return pl.BlockSpec((BR, D), i_map or (lambda i: (i, 0)))


def _full_spec(shape):
    return pl.BlockSpec(shape, lambda i: tuple(0 for _ in shape))


def _enc_body(x_ref, w0, b0, w1, b1, g, b, o_ref):
    x = x_ref[...]
    z = jnp.maximum(jnp.dot(x, w0[...], preferred_element_type=jnp.float32) + b0[...], 0.0)
    v = jnp.dot(z, w1[...], preferred_element_type=jnp.float32) + b1[...]
    o_ref[...] = _ln(v, g[...], b[...])


def _tc_encoder(x, w0, b0, w1, b1, g, b):
    rows = x.shape[0]
    grid = (rows // BR,)
    return pl.pallas_call(
        _enc_body,
        grid=grid,
        in_specs=[_row_spec(), _full_spec((D, D)), _full_spec((1, D)),
                  _full_spec((D, D)), _full_spec((1, D)),
                  _full_spec((1, D)), _full_spec((1, D))],
        out_specs=_row_spec(),
        out_shape=jax.ShapeDtypeStruct((rows, D), jnp.float32),
    )(x, w0, b0, w1, b1, g, b)


def _proj(h, wd, bd, ws, wh, bh):
    pd = jnp.dot(h, wd[...], preferred_element_type=jnp.float32) + bd[...]
    ps = jnp.dot(h, ws[...], preferred_element_type=jnp.float32)
    ph = jnp.dot(h, wh[...], preferred_element_type=jnp.float32) + bh[...]
    return pd, ps, ph


def _encp_body(x_ref, w0, b0, w1, b1, g, b, wd, bd, ws, wh, bh,
               h_ref, pd_ref, ps_ref, ph_ref):
    x = x_ref[...]
    z = jnp.maximum(jnp.dot(x, w0[...], preferred_element_type=jnp.float32) + b0[...], 0.0)
    v = jnp.dot(z, w1[...], preferred_element_type=jnp.float32) + b1[...]
    h = _ln(v, g[...], b[...])
    h_ref[...] = h
    pd_ref[...], ps_ref[...], ph_ref[...] = _proj(h, wd, bd, ws, wh, bh)


def _tc_enc_node_proj(x, w0, b0, w1, b1, g, b, wd, bd, ws, wh, bh):
    grid = (N // BR,)
    return pl.pallas_call(
        _encp_body,
        grid=grid,
        in_specs=[_row_spec(), _full_spec((D, D)), _full_spec((1, D)),
                  _full_spec((D, D)), _full_spec((1, D)),
                  _full_spec((1, D)), _full_spec((1, D)),
                  _full_spec((D, D)), _full_spec((1, D)),
                  _full_spec((D, D)), _full_spec((D, D)), _full_spec((1, D))],
        out_specs=[_row_spec()] * 4,
        out_shape=[jax.ShapeDtypeStruct((N, D), jnp.float32)] * 4,
    )(x, w0, b0, w1, b1, g, b, wd, bd, ws, wh, bh)


def _edge_body(gd_ref, gs_ref, e_ref, we, w2, b2, g, b, ne_ref, eo_ref):
    e = e_ref[...]
    z = gd_ref[...] + gs_ref[...] + jnp.dot(e, we[...], preferred_element_type=jnp.float32)
    u = jnp.maximum(z, 0.0)
    v = jnp.dot(u, w2[...], preferred_element_type=jnp.float32) + b2[...]
    ne = _ln(v, g[...], b[...])
    ne_ref[...] = ne
    eo_ref[...] = e + ne


def _tc_edge(gd, gs, e, we, w2, b2, g, b):
    grid = (E // BR,)
    return pl.pallas_call(
        _edge_body,
        grid=grid,
        in_specs=[_row_spec(), _row_spec(), _row_spec(),
                  _full_spec((D, D)), _full_spec((D, D)), _full_spec((1, D)),
                  _full_spec((1, D)), _full_spec((1, D))],
        out_specs=[_row_spec(), _row_spec()],
        out_shape=[jax.ShapeDtypeStruct((EP, D), jnp.float32),
                   jax.ShapeDtypeStruct((E, D), jnp.float32)],
    )(gd, gs, e, we, w2, b2, g, b)


def _upd(h_ref, ph_ref, agg_ref, wa, w2, b2, g, b):
    agg = agg_ref[0] + agg_ref[1]
    z = ph_ref[...] + jnp.dot(agg, wa[...], preferred_element_type=jnp.float32)
    u = jnp.maximum(z, 0.0)
    v = jnp.dot(u, w2[...], preferred_element_type=jnp.float32) + b2[...]
    return h_ref[...] + _ln(v, g[...], b[...])


def _updp_body(h_ref, ph_ref, agg_ref, wa, w2, b2, g, b, wd, bd, ws, wh, bh,
               ho_ref, pd_ref, ps_ref, ph2_ref):
    ho = _upd(h_ref, ph_ref, agg_ref, wa, w2, b2, g, b)
    ho_ref[...] = ho
    pd_ref[...], ps_ref[...], ph2_ref[...] = _proj(ho, wd, bd, ws, wh, bh)


def _tc_upd_proj(h, ph, aggp, wa, w2, b2, g, b, wd, bd, ws, wh, bh):
    grid = (N // BR,)
    return pl.pallas_call(
        _updp_body,
        grid=grid,
        in_specs=[_row_spec(), _row_spec(),
                  pl.BlockSpec((2, BR, D), lambda i: (0, i, 0)),
                  _full_spec((D, D)), _full_spec((D, D)), _full_spec((1, D)),
                  _full_spec((1, D)), _full_spec((1, D)),
                  _full_spec((D, D)), _full_spec((1, D)),
                  _full_spec((D, D)), _full_spec((D, D)), _full_spec((1, D))],
        out_specs=[_row_spec()] * 4,
        out_shape=[jax.ShapeDtypeStruct((N, D), jnp.float32)] * 4,
    )(h, ph, aggp, wa, w2, b2, g, b, wd, bd, ws, wh, bh)


def _upddec_body(h_ref, ph_ref, agg_ref, wa, w2, b2, g, b, dw0, db0, dw1, db1,
                 o_ref):
    ho = _upd(h_ref, ph_ref, agg_ref, wa, w2, b2, g, b)
    z = jnp.maximum(jnp.dot(ho, dw0[...], preferred_element_type=jnp.float32) + db0[...], 0.0)
    o_ref[...] = jnp.dot(z, dw1[...], preferred_element_type=jnp.float32) + db1[...]


def _tc_upd_dec(h, ph, aggp, wa, w2, b2, g, b, dw0, db0, dw1, db1):
    grid = (N // BR,)
    return pl.pallas_call(
        _upddec_body,
        grid=grid,
        in_specs=[_row_spec(), _row_spec(),
                  pl.BlockSpec((2, BR, D), lambda i: (0, i, 0)),
                  _full_spec((D, D)), _full_spec((D, D)), _full_spec((1, D)),
                  _full_spec((1, D)), _full_spec((1, D)),
                  _full_spec((D, D)), _full_spec((1, D)),
                  _full_spec((D, D)), _full_spec((1, D))],
        out_specs=_row_spec(),
        out_shape=jax.ShapeDtypeStruct((N, D), jnp.float32),
    )(h, ph, aggp, wa, w2, b2, g, b, dw0, db0, dw1, db1)


NBUF = 3
NBUF_S = 2


def _gather_body(pd, ps, dsti, srci, gd, gs, idxd, idxs,
                 bufsd, bufss, gsemd, gsems, wsemd, wsems):
    c = lax.axis_index("c")
    s = lax.axis_index("s")
    w = c * 16 + s
    row0 = w * CPW
    pltpu.sync_copy(dsti.at[pl.ds(row0, CPW)], idxd)
    pltpu.sync_copy(srci.at[pl.ds(row0, CPW)], idxs)

    def start_gather(j):
        p = j % NBUF
        return (pltpu.async_copy(pd.at[idxd.at[j]], bufsd.at[p], gsemd),
                pltpu.async_copy(ps.at[idxs.at[j]], bufss.at[p], gsems))

    def start_write(j):
        p = j % NBUF
        erow = pl.multiple_of((row0 + j) * CHUNK, CHUNK)
        return (pltpu.async_copy(bufsd.at[p], gd.at[pl.ds(erow, CHUNK)], wsemd),
                pltpu.async_copy(bufss.at[p], gs.at[pl.ds(erow, CHUNK)], wsems))

    g = [None] * CPW
    wr = [None] * CPW
    for j in range(NBUF):
        g[j] = start_gather(j)
    for j in range(CPW):
        g[j][0].wait()
        g[j][1].wait()
        wr[j] = start_write(j)
        if j + NBUF < CPW:
            wr[j][0].wait()
            wr[j][1].wait()
            g[j + NBUF] = start_gather(j + NBUF)
    for j in range(max(0, CPW - NBUF), CPW):
        wr[j][0].wait()
        wr[j][1].wait()


@functools.cache
def _sc_gather_kernel():
    return pl.kernel(
        _gather_body,
        out_type=[jax.ShapeDtypeStruct((EP, D), jnp.float32),
                  jax.ShapeDtypeStruct((EP, D), jnp.float32)],
        mesh=plsc.VectorSubcoreMesh(core_axis_name="c", subcore_axis_name="s"),
        scratch_types=[pltpu.VMEM((CPW, CHUNK), jnp.int32),
                       pltpu.VMEM((CPW, CHUNK), jnp.int32),
                       pltpu.VMEM((NBUF, CHUNK, D), jnp.float32),
                       pltpu.VMEM((NBUF, CHUNK, D), jnp.float32),
                       pltpu.SemaphoreType.DMA,
                       pltpu.SemaphoreType.DMA,
                       pltpu.SemaphoreType.DMA,
                       pltpu.SemaphoreType.DMA],
    )


def _sc_gather(pd, ps, dsti, srci):
    return _sc_gather_kernel()(pd, ps, dsti, srci)


def _scatter_body(ne, dsti, zz, out, idx, ebufs, acc, rsem, ssem):
    c = lax.axis_index("c")
    s = lax.axis_index("s")
    w = c * 16 + s
    row0 = w * CPW
    pltpu.sync_copy(zz.at[pl.ds(s * 632, 632)], acc.at[pl.ds(s * 632, 632)])
    pltpu.sync_copy(dsti.at[pl.ds(row0, CPW)], idx)
    plsc.subcore_barrier()

    def start_read(j):
        p = j % NBUF_S
        erow = pl.multiple_of((row0 + j) * CHUNK, CHUNK)
        return pltpu.async_copy(ne.at[pl.ds(erow, CHUNK)], ebufs.at[p], rsem)

    def start_scatter(j):
        p = j % NBUF_S
        return pltpu.async_copy(ebufs.at[p], acc.at[idx.at[j]], ssem, add=True)

    rd = [None] * CPW
    sc = [None] * CPW
    for j in range(NBUF_S):
        rd[j] = start_read(j)
    for j in range(CPW):
        rd[j].wait()
        sc[j] = start_scatter(j)
        if j + NBUF_S < CPW:
            sc[j].wait()
            rd[j + NBUF_S] = start_read(j + NBUF_S)
    for j in range(max(0, CPW - NBUF_S), CPW):
        sc[j].wait()
    plsc.subcore_barrier()

    @pl.when(s < 15)
    def _():
        pltpu.sync_copy(acc.at[pl.ds(s * 632, 632)], out.at[c, pl.ds(s * 632, 632)])

    @pl.when(s == 15)
    def _():
        pltpu.sync_copy(acc.at[pl.ds(15 * 632, N - 15 * 632)],
                        out.at[c, pl.ds(15 * 632, N - 15 * 632)])


@functools.cache
def _sc_scatter_kernel():
    return pl.kernel(
        _scatter_body,
        out_type=jax.ShapeDtypeStruct((2, N, D), jnp.float32),
        mesh=plsc.VectorSubcoreMesh(core_axis_name="c", subcore_axis_name="s"),
        scratch_types=[pltpu.VMEM((CPW, CHUNK), jnp.int32),
                       pltpu.VMEM((NBUF_S, CHUNK, D), jnp.float32),
                       pltpu.VMEM_SHARED((NP2, D), jnp.float32),
                       pltpu.SemaphoreType.DMA,
                       pltpu.SemaphoreType.DMA],
    )


def _sc_scatter(ne, dsti, zz):
    return _sc_scatter_kernel()(ne, dsti, zz)


def _r1(v):
    return v.reshape(1, D)


def kernel(x, edge_attr, edge_index, batch, params):
    src = edge_index[0]
    dst = edge_index[1]

    padz = jnp.zeros((EP - E,), jnp.int32)
    dst_g = jnp.concatenate([dst, padz]).reshape(EP // CHUNK, CHUNK)
    src_g = jnp.concatenate([src, padz]).reshape(EP // CHUNK, CHUNK)
    dst_s = jnp.concatenate([dst, jnp.full((EP - E,), PAD_SINK, jnp.int32)]
                            ).reshape(EP // CHUNK, CHUNK)
    zeros_acc = jnp.zeros((NP2, D), jnp.float32)

    def msg_split(blk):
        w1m = blk["msg"]["W"][0]
        return w1m[:D], _r1(blk["msg"]["b"][0]), w1m[D:2 * D], w1m[2 * D:]

    def upd_split(blk):
        w1u = blk["upd"]["W"][0]
        return w1u[:D], _r1(blk["upd"]["b"][0]), w1u[D:]

    blocks = params["blocks"]
    en = params["enc_node"]
    ee = params["enc_edge"]
    e = _tc_encoder(edge_attr, ee["W"][0], _r1(ee["b"][0]), ee["W"][1], _r1(ee["b"][1]),
                    _r1(ee["ln_g"]), _r1(ee["ln_b"]))

    wd, bd, ws, we = msg_split(blocks[0])
    wh, bh, wa = upd_split(blocks[0])
    h, pd, ps, ph = _tc_enc_node_proj(
        x, en["W"][0], _r1(en["b"][0]), en["W"][1], _r1(en["b"][1]),
        _r1(en["ln_g"]), _r1(en["ln_b"]), wd, bd, ws, wh, bh)

    dec = params["dec"]
    dw0 = jnp.pad(dec["W"][0], ((0, 0), (0, D - OUT)))
    db0 = _r1(jnp.pad(dec["b"][0], (0, D - OUT)))
    dw1 = jnp.pad(dec["W"][1], ((0, D - OUT), (0, D - OUT)))
    db1 = _r1(jnp.pad(dec["b"][1], (0, D - OUT)))

    for k, blk in enumerate(blocks):
        m, u = blk["msg"], blk["upd"]
        gd, gs = _sc_gather(pd, ps, dst_g, src_g)
        ne, e = _tc_edge(gd, gs, e, we, m["W"][1], _r1(m["b"][1]),
                         _r1(m["ln_g"]), _r1(m["ln_b"]))
        aggp = _sc_scatter(ne, dst_s, zeros_acc)
        if k + 1 < len(blocks):
            wd, bd, ws, we = msg_split(blocks[k + 1])
            wh2, bh2, wa2 = upd_split(blocks[k + 1])
            h, pd, ps, ph = _tc_upd_proj(
                h, ph, aggp, wa, u["W"][1], _r1(u["b"][1]),
                _r1(u["ln_g"]), _r1(u["ln_b"]), wd, bd, ws, wh2, bh2)
            wa = wa2
        else:
            out_full = _tc_upd_dec(
                h, ph, aggp, wa, u["W"][1], _r1(u["b"][1]),
                _r1(u["ln_g"]), _r1(u["ln_b"]), dw0, db0, dw1, db1)
    return out_full[:, :OUT]
